# baseline (device time: 62737 ns/iter reference)
import jax
import jax.numpy as jnp
from jax import lax
from jax.experimental import pallas as pl
from jax.experimental.pallas import tpu as pltpu

TC = 32


def kernel(x, A, B, C):
    b, s_loc, d = x.shape
    n = A.shape[1]
    n_chunks = s_loc // TC

    def body(x_ref, a_ref, b_ref, c_ref, xt_ref, bt_ref, out_ref,
             sbuf, rbuf, carry, send_sem, recv_sem):
        i = pl.program_id(0)
        my_x = lax.axis_index("x")
        my_y = lax.axis_index("y")
        partner = (1 - my_x, my_y)

        dAt = jnp.exp(a_ref[...]).T

        @pl.when(i == 0)
        def _():
            barrier_sem = pltpu.get_barrier_semaphore()
            pl.semaphore_signal(
                barrier_sem, inc=1,
                device_id=partner, device_id_type=pl.DeviceIdType.MESH,
            )
            pl.semaphore_wait(barrier_sem, 1)

            h = jnp.zeros((b, n, d), jnp.float32)
            for s in range(TC):
                xs = xt_ref[:, s, :]
                bs = bt_ref[:, s, :]
                h = h * dAt[None] + xs[:, None, :] * bs[:, :, None]
            sbuf[...] = h

            copy = pltpu.make_async_remote_copy(
                src_ref=sbuf, dst_ref=rbuf,
                send_sem=send_sem, recv_sem=recv_sem,
                device_id=partner, device_id_type=pl.DeviceIdType.MESH,
            )

            @pl.when(my_x == 0)
            def _():
                copy.start()
                copy.wait_send()

            @pl.when(my_x == 1)
            def _():
                copy.wait_recv()

            carry[...] = jnp.where(my_x == 1, rbuf[...], 0.0)

        h = carry[...]
        for s in range(TC):
            xs = x_ref[:, s, :]
            bs = b_ref[:, s, :]
            cs = c_ref[:, s, :]
            h = h * dAt[None] + xs[:, None, :] * bs[:, :, None]
            out_ref[:, s, :] = jnp.sum(h * cs[:, :, None], axis=1)
        carry[...] = h

        @pl.when(i == n_chunks - 1)
        def _():
            def exit_barrier(sem):
                pl.semaphore_signal(
                    sem, inc=1,
                    device_id=partner, device_id_type=pl.DeviceIdType.MESH,
                )
                pl.semaphore_wait(sem, 1)
            pl.run_scoped(exit_barrier, pltpu.SemaphoreType.REGULAR)

    return pl.pallas_call(
        body,
        grid=(n_chunks,),
        in_specs=[
            pl.BlockSpec((b, TC, d), lambda i: (0, i, 0)),
            pl.BlockSpec(memory_space=pltpu.VMEM),
            pl.BlockSpec((b, TC, n), lambda i: (0, i, 0)),
            pl.BlockSpec((b, TC, n), lambda i: (0, i, 0)),
            pl.BlockSpec((b, TC, d), lambda i: (0, n_chunks - 1, 0)),
            pl.BlockSpec((b, TC, n), lambda i: (0, n_chunks - 1, 0)),
        ],
        out_specs=pl.BlockSpec((b, TC, d), lambda i: (0, i, 0)),
        out_shape=jax.ShapeDtypeStruct((b, s_loc, d), jnp.float32),
        scratch_shapes=[
            pltpu.VMEM((b, n, d), jnp.float32),
            pltpu.VMEM((b, n, d), jnp.float32),
            pltpu.VMEM((b, n, d), jnp.float32),
            pltpu.SemaphoreType.DMA,
            pltpu.SemaphoreType.DMA,
        ],
        compiler_params=pltpu.CompilerParams(
            collective_id=0,
            dimension_semantics=("arbitrary",),
        ),
    )(x, A, B, C, x, B)
